# baseline (speedup 1.0000x reference)
import functools

import jax
import jax.numpy as jnp
from jax.experimental import pallas as pl
from jax.experimental.pallas import tpu as pltpu

D_MODEL_ = 4096
N_EXPERTS_ = 8
N_MODULES_ = 4
K_TOP_ = 2
B_ = 8192

B_T = 1024
C_K = 512
N_BT = B_ // B_T
N_KC = D_MODEL_ // C_K


def _router_kernel(p_ref, wg_ref, wr_ref, q_ref, k_ref, v_ref, o_ref,
                   acc_ref, *, precision):
    kc = pl.program_id(1)

    @pl.when(kc == 0)
    def _():
        acc_ref[...] = jnp.zeros_like(acc_ref)

    gated = jax.lax.dot_general(
        p_ref[...], wg_ref[...], (((1,), (1,)), ((), ())),
        precision=precision, preferred_element_type=jnp.float32)
    acc_ref[...] += jax.lax.dot_general(
        gated, wr_ref[...], (((1,), (1,)), ((), ())),
        precision=precision, preferred_element_type=jnp.float32)

    @pl.when(kc == N_KC - 1)
    def _():
        logits = acc_ref[...]
        m = jnp.max(logits, axis=-1, keepdims=True)
        e = jnp.exp(logits - m)
        denom = jnp.sum(e, axis=-1, keepdims=True)
        probs = e / denom
        col = jax.lax.broadcasted_iota(jnp.int32, (B_T, N_EXPERTS_), 1)
        hi = jnp.where(col < K_TOP_, 1.0 / K_TOP_, 0.0).astype(jnp.float32)
        lo = jnp.where(col < 1, 1.0, 0.0).astype(jnp.float32)
        for i, ref in enumerate((q_ref, k_ref, v_ref, o_ref)):
            sel = probs[:, i:i + 1] > 0.5
            ref[...] = jnp.where(sel, hi, lo)


def _make_call(precision):
    kfn = functools.partial(_router_kernel, precision=precision)
    out_spec = pl.BlockSpec((B_T, N_EXPERTS_), lambda i, k: (i, 0))
    return pl.pallas_call(
        kfn,
        grid=(N_BT, N_KC),
        in_specs=[
            pl.BlockSpec((B_T, D_MODEL_), lambda i, k: (i, 0)),
            pl.BlockSpec((C_K, D_MODEL_), lambda i, k: (k, 0)),
            pl.BlockSpec((N_MODULES_, C_K), lambda i, k: (0, k)),
        ],
        out_specs=[out_spec] * N_MODULES_,
        out_shape=[jax.ShapeDtypeStruct((B_, N_EXPERTS_), jnp.float32)] * N_MODULES_,
        scratch_shapes=[pltpu.VMEM((B_T, N_MODULES_), jnp.float32)],
        compiler_params=pltpu.CompilerParams(
            dimension_semantics=("parallel", "arbitrary"),
        ),
    )


def kernel(pooled_hidden, Wg, Wr):
    call = _make_call(jax.lax.Precision.DEFAULT)
    q, k, v, o = call(pooled_hidden, Wg, Wr)
    return (q, k, v, o)


# speedup vs baseline: 1.0009x; 1.0009x over previous
import jax
import jax.numpy as jnp
from jax.experimental import pallas as pl
from jax.experimental.pallas import tpu as pltpu

D_MODEL_ = 4096
N_EXPERTS_ = 8
N_MODULES_ = 4
K_TOP_ = 2
B_ = 8192

B_T = 1024
C_K = 512
N_BT = B_ // B_T
N_KC = D_MODEL_ // C_K

_PREC = jax.lax.Precision.DEFAULT


def _router_kernel(p_ref, wg_ref, wr_ref, q_ref, k_ref, v_ref, o_ref,
                   acc_ref):
    kc = pl.program_id(1)

    @pl.when(kc == 0)
    def _():
        acc_ref[...] = jnp.zeros_like(acc_ref)

    gated = jax.lax.dot_general(
        p_ref[...], wg_ref[...], (((1,), (1,)), ((), ())),
        precision=_PREC, preferred_element_type=jnp.float32)
    acc_ref[...] += jax.lax.dot_general(
        gated, wr_ref[...], (((1,), (1,)), ((), ())),
        precision=_PREC, preferred_element_type=jnp.float32)

    @pl.when(kc == N_KC - 1)
    def _():
        logits = acc_ref[...]
        m = jnp.max(logits, axis=-1, keepdims=True)
        e = jnp.exp(logits - m)
        denom = jnp.sum(e, axis=-1, keepdims=True)
        probs = e / denom
        col = jax.lax.broadcasted_iota(jnp.int32, (B_T, N_EXPERTS_), 1)
        hi = jnp.where(col < K_TOP_, 1.0 / K_TOP_, 0.0).astype(jnp.float32)
        lo = jnp.where(col < 1, 1.0, 0.0).astype(jnp.float32)
        for i, ref in enumerate((q_ref, k_ref, v_ref, o_ref)):
            sel = probs[:, i:i + 1] > 0.5
            ref[...] = jnp.where(sel, hi, lo)


def _make_call():
    out_spec = pl.BlockSpec((B_T, N_EXPERTS_), lambda i, k: (i, 0))
    return pl.pallas_call(
        _router_kernel,
        grid=(N_BT, N_KC),
        in_specs=[
            pl.BlockSpec((B_T, D_MODEL_), lambda i, k: (i, 0)),
            pl.BlockSpec((C_K, D_MODEL_), lambda i, k: (k, 0)),
            pl.BlockSpec((N_MODULES_, C_K), lambda i, k: (0, k)),
        ],
        out_specs=[out_spec] * N_MODULES_,
        out_shape=[jax.ShapeDtypeStruct((B_, N_EXPERTS_), jnp.float32)] * N_MODULES_,
        scratch_shapes=[pltpu.VMEM((B_T, N_MODULES_), jnp.float32)],
        compiler_params=pltpu.CompilerParams(
            dimension_semantics=("parallel", "arbitrary"),
        ),
    )


def kernel(pooled_hidden, Wg, Wr):
    q, k, v, o = _make_call()(pooled_hidden, Wg, Wr)
    return (q, k, v, o)
